# baseline (speedup 1.0000x reference)
import numpy as np
import jax
import jax.numpy as jnp
from jax import lax
from jax.experimental import pallas as pl
from jax.experimental.pallas import tpu as pltpu
from jax.experimental.pallas import tpu_sc as plsc

N = 10000
E = 160000
D1 = 256
D2 = 512

NC = 2
NS = 16
HALF = N // 2

BW = 5
BLK = BW * 128
NBLK = E // BLK
NACC = 5120
OCH = NACC // 128
JUNK = 128
NP = 10240

_MESH = plsc.VectorSubcoreMesh(core_axis_name="c", subcore_axis_name="s")
_SCPARAMS = pltpu.CompilerParams(needs_layout_passes=False,
                                 use_tc_tiling_on_sc=False)


def _deg_body(dstf, zdeg, out_hbm, dbuf, dacc, comb, outb, sp, sem):
    c = lax.axis_index("c")
    s = lax.axis_index("s")
    ones = jnp.full((16,), 1.0, jnp.float32)
    lane0 = lax.iota(jnp.int32, 16) == 0

    pltpu.sync_copy(zdeg, dacc)

    nrows = jnp.where(s < 2, 79, 78)
    rstart = s * 78 + jnp.minimum(s, 2)

    def row(i, _):
        off = pl.multiple_of((rstart + i) * 128, 8)
        pltpu.sync_copy(dstf.at[pl.ds(off, 128)], dbuf)
        for q in range(8):
            v = dbuf[pl.ds(q * 16, 16)]
            for j in range(16):
                rid = jnp.take(v, jnp.full((16,), j, jnp.int32))
                plsc.addupdate_scatter(dacc, [rid], ones, mask=lane0)
        return 0
    lax.fori_loop(0, nrows, row, 0)

    pltpu.sync_copy(dacc, sp.at[s])
    plsc.subcore_barrier()

    @pl.when(c == 0)
    def _():
        for t in range(NS):
            pltpu.sync_copy(sp.at[t, pl.ds(s * 640, 640)], comb.at[t])
        for j in range(640 // 16):
            tot = comb[0, pl.ds(j * 16, 16)]
            for t in range(1, NS):
                tot = tot + comb[t, pl.ds(j * 16, 16)]
            outb[pl.ds(j * 16, 16)] = tot
        pltpu.sync_copy(outb, out_hbm.at[pl.ds(s * 640, 640)])


def _deg_call(dstf, zdeg):
    fn = pl.kernel(
        _deg_body,
        out_type=jax.ShapeDtypeStruct((NP,), jnp.float32),
        mesh=_MESH,
        compiler_params=_SCPARAMS,
        scratch_types=[
            pltpu.VMEM((128,), jnp.int32),
            pltpu.VMEM((NP,), jnp.float32),
            pltpu.VMEM((NS, 640), jnp.float32),
            pltpu.VMEM((640,), jnp.float32),
            pltpu.VMEM_SHARED((NS, NP), jnp.float32),
            pltpu.SemaphoreType.DMA,
        ],
    )
    return fn(dstf, zdeg)


def _make_agg_body(FH, ROUNDS):
    def body(srcf, dstf, featf, zacc, oidx_hbm, out_hbm,
             sbufA, dbufA, gbufA, sbufB, dbufB, gbufB,
             oidx, rows, acc, semS, semG):
        c = lax.axis_index("c")
        s = lax.axis_index("s")
        w = s * NC + c
        iota = lax.iota(jnp.int32, 16)
        jsel = [jnp.full((16,), j, jnp.int32) for j in range(16)]

        def stage(blk, sbuf, dbuf):
            off = pl.multiple_of(blk * BLK, 8)
            pltpu.async_copy(srcf.at[pl.ds(off, BLK)], sbuf, semS)
            pltpu.async_copy(dstf.at[pl.ds(off, BLK)], dbuf, semS)

        def drain_stage(sbuf, dbuf):
            pltpu.make_async_copy(srcf.at[pl.ds(0, BLK)], sbuf, semS).wait()
            pltpu.make_async_copy(dstf.at[pl.ds(0, BLK)], dbuf, semS).wait()

        def round_body(rnd, _):
            task = rnd * (NC * NS) + w
            m = lax.rem(task, FH)
            h = lax.div(task, FH)
            basev = jnp.full((16,), h * HALF, jnp.int32)
            clampv = jnp.full((16,), NACC - 1, jnp.uint32)

            pltpu.sync_copy(zacc, acc)
            pltpu.sync_copy(oidx_hbm.at[pl.ds(task * OCH, OCH)], oidx)

            def process(sbuf, dbuf, gbuf, nxt_blk, nxt_s, nxt_d):
                drain_stage(sbuf, dbuf)
                for i in range(BW * 8):
                    v = sbuf[pl.ds(i * 16, 16)]
                    gbuf[pl.ds(i * 16, 16)] = v * FH + m
                descs = []
                for k in range(BW):
                    descs.append(pltpu.async_copy(
                        featf.at[gbuf.at[pl.ds(k * 128, 64)]],
                        rows.at[k, pl.ds(0, 64)], semG))
                    descs.append(pltpu.async_copy(
                        featf.at[gbuf.at[pl.ds(k * 128 + 64, 64)]],
                        rows.at[k, pl.ds(64, 64)], semG))
                stage(jnp.minimum(nxt_blk, NBLK - 1), nxt_s, nxt_d)
                for k in range(BW):
                    descs[2 * k].wait()
                    descs[2 * k + 1].wait()
                    for q in range(8):
                        dv = dbuf[pl.ds((k * 8 + q) * 16, 16)]
                        rloc = lax.bitcast_convert_type(dv - basev, jnp.uint32)
                        rloc = lax.bitcast_convert_type(
                            jnp.minimum(rloc, clampv), jnp.int32)
                        for j0 in range(0, 16, 8):
                            rids = [jnp.take(rloc, jsel[j0 + t])
                                    for t in range(8)]
                            vals = [rows[k, q * 16 + j0 + t, pl.ds(0, 16)]
                                    for t in range(8)]
                            for t in range(8):
                                plsc.addupdate_scatter(acc, [rids[t], iota],
                                                       vals[t])

            stage(0, sbufA, dbufA)

            def pair(i, _):
                process(sbufA, dbufA, gbufA, 2 * i + 1, sbufB, dbufB)
                process(sbufB, dbufB, gbufB, 2 * i + 2, sbufA, dbufA)
                return 0

            lax.fori_loop(0, NBLK // 2, pair, 0)
            drain_stage(sbufA, dbufA)

            odescs = []
            for ch in range(OCH):
                odescs.append(pltpu.async_copy(
                    acc.at[pl.ds(ch * 128, 128)],
                    out_hbm.at[oidx.at[ch]], semG))
            for d in odescs:
                d.wait()
            return 0

        lax.fori_loop(0, ROUNDS, round_body, 0)

    return body


def _agg_call(srcf, dstf, featf, zacc, oidx_all, FH, ROUNDS):
    body = _make_agg_body(FH, ROUNDS)
    fn = pl.kernel(
        body,
        out_type=jax.ShapeDtypeStruct((N * FH + JUNK, 16), jnp.float32),
        mesh=_MESH,
        compiler_params=_SCPARAMS,
        scratch_types=[
            pltpu.VMEM((BLK,), jnp.int32),
            pltpu.VMEM((BLK,), jnp.int32),
            pltpu.VMEM((BLK,), jnp.int32),
            pltpu.VMEM((BLK,), jnp.int32),
            pltpu.VMEM((BLK,), jnp.int32),
            pltpu.VMEM((BLK,), jnp.int32),
            pltpu.VMEM((OCH, 128), jnp.int32),
            pltpu.VMEM((BW, 128, 16), jnp.float32),
            pltpu.VMEM((NACC, 16), jnp.float32),
            pltpu.SemaphoreType.DMA,
            pltpu.SemaphoreType.DMA,
        ],
    )
    return fn(srcf, dstf, featf, zacc, oidx_all)


def _oidx_table(FH):
    tasks = FH * 2
    t = np.arange(NACC, dtype=np.int64)
    tables = np.empty((tasks, NACC), dtype=np.int32)
    for task in range(tasks):
        m = task % FH
        h = task // FH
        node = h * HALF + t
        main = node * FH + m
        junk = N * FH + (t - HALF)
        tables[task] = np.where(t < HALF, main, junk).astype(np.int32)
    return jnp.asarray(tables.reshape(tasks * OCH, 128))


BN = 400
GRID = N // BN


def _scale_body(deg_ref, x_ref, xs_ref, disc_ref):
    dis = lax.rsqrt(deg_ref[...] + 1.0)
    disc_ref[...] = dis
    xs_ref[...] = x_ref[...] * dis


def _scale_call(degc, x):
    return pl.pallas_call(
        _scale_body,
        grid=(GRID,),
        in_specs=[
            pl.BlockSpec((BN, 1), lambda j: (j, 0)),
            pl.BlockSpec((BN, D1), lambda j: (j, 0)),
        ],
        out_specs=[
            pl.BlockSpec((BN, D1), lambda j: (j, 0)),
            pl.BlockSpec((BN, 1), lambda j: (j, 0)),
        ],
        out_shape=[
            jax.ShapeDtypeStruct((N, D1), jnp.float32),
            jax.ShapeDtypeStruct((N, 1), jnp.float32),
        ],
    )(degc, x)


def _mm1_body(agg_ref, xs_ref, disc_ref, We_ref, W1_ref, b1_ref, h1s_ref):
    dcol = disc_ref[...]
    z = (agg_ref[...] + xs_ref[...]) * dcol
    t = lax.dot_general(z, We_ref[...], (((1,), (1,)), ((), ())),
                        preferred_element_type=jnp.float32)
    h = lax.dot_general(t, W1_ref[...], (((1,), (1,)), ((), ())),
                        preferred_element_type=jnp.float32)
    h = jnp.maximum(h + b1_ref[...], 0.0)
    h1s_ref[...] = h * dcol


def _mm1_call(agg1, xs, disc, We, W1, b1r):
    return pl.pallas_call(
        _mm1_body,
        grid=(GRID,),
        in_specs=[
            pl.BlockSpec((BN, D1), lambda j: (j, 0)),
            pl.BlockSpec((BN, D1), lambda j: (j, 0)),
            pl.BlockSpec((BN, 1), lambda j: (j, 0)),
            pl.BlockSpec((D2, D1), lambda j: (0, 0)),
            pl.BlockSpec((D2, D2), lambda j: (0, 0)),
            pl.BlockSpec((1, D2), lambda j: (0, 0)),
        ],
        out_specs=pl.BlockSpec((BN, D2), lambda j: (j, 0)),
        out_shape=jax.ShapeDtypeStruct((N, D2), jnp.float32),
    )(agg1, xs, disc, We, W1, b1r)


def _mm2_body(agg_ref, h1s_ref, disc_ref, W2_ref, b2_ref, Wc1_ref, bc1_ref,
              Wc2_ref, bc2_ref, h2_ref, log_ref):
    dcol = disc_ref[...]
    z = (agg_ref[...] + h1s_ref[...]) * dcol
    h2 = lax.dot_general(z, W2_ref[...], (((1,), (1,)), ((), ())),
                         preferred_element_type=jnp.float32) + b2_ref[...]
    h2_ref[...] = h2
    cc = lax.dot_general(h2, Wc1_ref[...], (((1,), (1,)), ((), ())),
                         preferred_element_type=jnp.float32)
    cc = jnp.maximum(cc + bc1_ref[...], 0.0)
    log_ref[...] = lax.dot_general(cc, Wc2_ref[...], (((1,), (1,)), ((), ())),
                                   preferred_element_type=jnp.float32) + bc2_ref[...]


def _mm2_call(agg2, h1s, disc, W2, b2r, Wc1, bc1r, Wc2p, bc2p):
    HC = D2 // 2
    return pl.pallas_call(
        _mm2_body,
        grid=(GRID,),
        in_specs=[
            pl.BlockSpec((BN, D2), lambda j: (j, 0)),
            pl.BlockSpec((BN, D2), lambda j: (j, 0)),
            pl.BlockSpec((BN, 1), lambda j: (j, 0)),
            pl.BlockSpec((D2, D2), lambda j: (0, 0)),
            pl.BlockSpec((1, D2), lambda j: (0, 0)),
            pl.BlockSpec((HC, D2), lambda j: (0, 0)),
            pl.BlockSpec((1, HC), lambda j: (0, 0)),
            pl.BlockSpec((16, HC), lambda j: (0, 0)),
            pl.BlockSpec((1, 16), lambda j: (0, 0)),
        ],
        out_specs=[
            pl.BlockSpec((BN, D2), lambda j: (j, 0)),
            pl.BlockSpec((BN, 16), lambda j: (j, 0)),
        ],
        out_shape=[
            jax.ShapeDtypeStruct((N, D2), jnp.float32),
            jax.ShapeDtypeStruct((N, 16), jnp.float32),
        ],
    )(agg2, h1s, disc, W2, b2r, Wc1, bc1r, Wc2p, bc2p)


@jax.jit
def kernel(x, edge_index, We, be, W1, b1, W2, b2, Wc1, bc1, Wc2, bc2):
    srcf = edge_index[0]
    dstf = edge_index[1]

    deg = _deg_call(dstf, jnp.zeros((NP,), jnp.float32))
    xs, disc = _scale_call(deg[:N].reshape(N, 1), x)

    FH1 = D1 // 16
    agg1f = _agg_call(srcf, dstf, xs.reshape(N * FH1, 16),
                      jnp.zeros((NACC, 16), jnp.float32),
                      _oidx_table(FH1), FH1, 1)
    agg1 = agg1f[:N * FH1].reshape(N, D1)

    h1s = _mm1_call(agg1, xs, disc, We, W1, b1.reshape(1, D2))

    FH2 = D2 // 16
    agg2f = _agg_call(srcf, dstf, h1s.reshape(N * FH2, 16),
                      jnp.zeros((NACC, 16), jnp.float32),
                      _oidx_table(FH2), FH2, 2)
    agg2 = agg2f[:N * FH2].reshape(N, D2)

    Wc2p = jnp.zeros((16, D2 // 2), jnp.float32).at[:10].set(Wc2)
    bc2p = jnp.zeros((1, 16), jnp.float32).at[:, :10].set(bc2)
    h2, logits16 = _mm2_call(agg2, h1s, disc, W2, b2.reshape(1, D2),
                             Wc1, bc1.reshape(1, D2 // 2), Wc2p, bc2p)
    return logits16[:, :10], h2


# speedup vs baseline: 1.0181x; 1.0181x over previous
import numpy as np
import jax
import jax.numpy as jnp
from jax import lax
from jax.experimental import pallas as pl
from jax.experimental.pallas import tpu as pltpu
from jax.experimental.pallas import tpu_sc as plsc

N = 10000
E = 160000
D1 = 256
D2 = 512

NC = 2
NS = 16
HALF = N // 2

BW = 5
BLK = BW * 128
NBLK = E // BLK
NACC = 5120
OCH = NACC // 128
JUNK = 128
NP = 10240

_MESH = plsc.VectorSubcoreMesh(core_axis_name="c", subcore_axis_name="s")
_SCPARAMS = pltpu.CompilerParams(needs_layout_passes=False,
                                 use_tc_tiling_on_sc=False)


def _deg_body(dstf, zdeg, out_hbm, dbuf, dacc, comb, outb, sp, sem):
    c = lax.axis_index("c")
    s = lax.axis_index("s")
    ones = jnp.full((16,), 1.0, jnp.float32)
    lane0 = lax.iota(jnp.int32, 16) == 0

    pltpu.sync_copy(zdeg, dacc)

    nrows = jnp.where(s < 2, 79, 78)
    rstart = s * 78 + jnp.minimum(s, 2)

    def row(i, _):
        off = pl.multiple_of((rstart + i) * 128, 8)
        pltpu.sync_copy(dstf.at[pl.ds(off, 128)], dbuf)
        for q in range(8):
            v = dbuf[pl.ds(q * 16, 16)]
            for j in range(16):
                rid = jnp.take(v, jnp.full((16,), j, jnp.int32))
                plsc.addupdate_scatter(dacc, [rid], ones, mask=lane0)
        return 0
    lax.fori_loop(0, nrows, row, 0)

    pltpu.sync_copy(dacc, sp.at[s])
    plsc.subcore_barrier()

    @pl.when(c == 0)
    def _():
        for t in range(NS):
            pltpu.sync_copy(sp.at[t, pl.ds(s * 640, 640)], comb.at[t])
        for j in range(640 // 16):
            tot = comb[0, pl.ds(j * 16, 16)]
            for t in range(1, NS):
                tot = tot + comb[t, pl.ds(j * 16, 16)]
            outb[pl.ds(j * 16, 16)] = tot
        pltpu.sync_copy(outb, out_hbm.at[pl.ds(s * 640, 640)])


def _deg_call(dstf, zdeg):
    fn = pl.kernel(
        _deg_body,
        out_type=jax.ShapeDtypeStruct((NP,), jnp.float32),
        mesh=_MESH,
        compiler_params=_SCPARAMS,
        scratch_types=[
            pltpu.VMEM((128,), jnp.int32),
            pltpu.VMEM((NP,), jnp.float32),
            pltpu.VMEM((NS, 640), jnp.float32),
            pltpu.VMEM((640,), jnp.float32),
            pltpu.VMEM_SHARED((NS, NP), jnp.float32),
            pltpu.SemaphoreType.DMA,
        ],
    )
    return fn(dstf, zdeg)


def _make_agg_body(FH, ROUNDS):
    def body(srcf, dstf, featf, zacc, oidx_hbm, out_hbm,
             sbufA, dbufA, gbufA, sbufB, dbufB, gbufB,
             oidx, rows, acc, semS, semG):
        c = lax.axis_index("c")
        s = lax.axis_index("s")
        w = s * NC + c
        iota = lax.iota(jnp.int32, 16)
        jsel = [jnp.full((16,), j, jnp.int32) for j in range(16)]

        def stage(blk, sbuf, dbuf):
            off = pl.multiple_of(blk * BLK, 8)
            pltpu.async_copy(srcf.at[pl.ds(off, BLK)], sbuf, semS)
            pltpu.async_copy(dstf.at[pl.ds(off, BLK)], dbuf, semS)

        def drain_stage(sbuf, dbuf):
            pltpu.make_async_copy(srcf.at[pl.ds(0, BLK)], sbuf, semS).wait()
            pltpu.make_async_copy(dstf.at[pl.ds(0, BLK)], dbuf, semS).wait()

        def round_body(rnd, _):
            task = rnd * (NC * NS) + w
            m = lax.rem(task, FH)
            h = lax.div(task, FH)
            basev = jnp.full((16,), h * HALF, jnp.int32)
            clampv = jnp.full((16,), NACC - 1, jnp.uint32)

            pltpu.sync_copy(zacc, acc)
            pltpu.sync_copy(oidx_hbm.at[pl.ds(task * OCH, OCH)], oidx)

            def process(sbuf, dbuf, gbuf, nxt_blk, nxt_s, nxt_d):
                drain_stage(sbuf, dbuf)
                for i in range(BW * 8):
                    v = sbuf[pl.ds(i * 16, 16)]
                    gbuf[pl.ds(i * 16, 16)] = v * FH + m
                descs = []
                for k in range(BW):
                    descs.append(pltpu.async_copy(
                        featf.at[gbuf.at[pl.ds(k * 128, 128)]],
                        rows.at[k], semG))
                stage(jnp.minimum(nxt_blk, NBLK - 1), nxt_s, nxt_d)
                for k in range(BW):
                    descs[k].wait()
                    for q in range(8):
                        dv = dbuf[pl.ds((k * 8 + q) * 16, 16)]
                        rloc = lax.bitcast_convert_type(dv - basev, jnp.uint32)
                        rloc = lax.bitcast_convert_type(
                            jnp.minimum(rloc, clampv), jnp.int32)
                        for j0 in range(0, 16, 8):
                            rids = [jnp.take(rloc, jsel[j0 + t])
                                    for t in range(8)]
                            vals = [rows[k, q * 16 + j0 + t, pl.ds(0, 16)]
                                    for t in range(8)]
                            for t in range(8):
                                plsc.addupdate_scatter(acc, [rids[t], iota],
                                                       vals[t])

            stage(0, sbufA, dbufA)

            def pair(i, _):
                process(sbufA, dbufA, gbufA, 2 * i + 1, sbufB, dbufB)
                process(sbufB, dbufB, gbufB, 2 * i + 2, sbufA, dbufA)
                return 0

            lax.fori_loop(0, NBLK // 2, pair, 0)
            drain_stage(sbufA, dbufA)

            odescs = []
            for ch in range(OCH):
                odescs.append(pltpu.async_copy(
                    acc.at[pl.ds(ch * 128, 128)],
                    out_hbm.at[oidx.at[ch]], semG))
            for d in odescs:
                d.wait()
            return 0

        lax.fori_loop(0, ROUNDS, round_body, 0)

    return body


def _agg_call(srcf, dstf, featf, zacc, oidx_all, FH, ROUNDS):
    body = _make_agg_body(FH, ROUNDS)
    fn = pl.kernel(
        body,
        out_type=jax.ShapeDtypeStruct((N * FH + JUNK, 16), jnp.float32),
        mesh=_MESH,
        compiler_params=_SCPARAMS,
        scratch_types=[
            pltpu.VMEM((BLK,), jnp.int32),
            pltpu.VMEM((BLK,), jnp.int32),
            pltpu.VMEM((BLK,), jnp.int32),
            pltpu.VMEM((BLK,), jnp.int32),
            pltpu.VMEM((BLK,), jnp.int32),
            pltpu.VMEM((BLK,), jnp.int32),
            pltpu.VMEM((OCH, 128), jnp.int32),
            pltpu.VMEM((BW, 128, 16), jnp.float32),
            pltpu.VMEM((NACC, 16), jnp.float32),
            pltpu.SemaphoreType.DMA,
            pltpu.SemaphoreType.DMA,
        ],
    )
    return fn(srcf, dstf, featf, zacc, oidx_all)


def _oidx_table(FH):
    tasks = FH * 2
    t = np.arange(NACC, dtype=np.int64)
    tables = np.empty((tasks, NACC), dtype=np.int32)
    for task in range(tasks):
        m = task % FH
        h = task // FH
        node = h * HALF + t
        main = node * FH + m
        junk = N * FH + (t - HALF)
        tables[task] = np.where(t < HALF, main, junk).astype(np.int32)
    return jnp.asarray(tables.reshape(tasks * OCH, 128))


BN = 400
GRID = N // BN


def _scale_body(deg_ref, x_ref, xs_ref, disc_ref):
    dis = lax.rsqrt(deg_ref[...] + 1.0)
    disc_ref[...] = dis
    xs_ref[...] = x_ref[...] * dis


def _scale_call(degc, x):
    return pl.pallas_call(
        _scale_body,
        grid=(GRID,),
        in_specs=[
            pl.BlockSpec((BN, 1), lambda j: (j, 0)),
            pl.BlockSpec((BN, D1), lambda j: (j, 0)),
        ],
        out_specs=[
            pl.BlockSpec((BN, D1), lambda j: (j, 0)),
            pl.BlockSpec((BN, 1), lambda j: (j, 0)),
        ],
        out_shape=[
            jax.ShapeDtypeStruct((N, D1), jnp.float32),
            jax.ShapeDtypeStruct((N, 1), jnp.float32),
        ],
    )(degc, x)


def _mm1_body(agg_ref, xs_ref, disc_ref, We_ref, W1_ref, b1_ref, h1s_ref):
    dcol = disc_ref[...]
    z = (agg_ref[...] + xs_ref[...]) * dcol
    t = lax.dot_general(z, We_ref[...], (((1,), (1,)), ((), ())),
                        preferred_element_type=jnp.float32)
    h = lax.dot_general(t, W1_ref[...], (((1,), (1,)), ((), ())),
                        preferred_element_type=jnp.float32)
    h = jnp.maximum(h + b1_ref[...], 0.0)
    h1s_ref[...] = h * dcol


def _mm1_call(agg1, xs, disc, We, W1, b1r):
    return pl.pallas_call(
        _mm1_body,
        grid=(GRID,),
        in_specs=[
            pl.BlockSpec((BN, D1), lambda j: (j, 0)),
            pl.BlockSpec((BN, D1), lambda j: (j, 0)),
            pl.BlockSpec((BN, 1), lambda j: (j, 0)),
            pl.BlockSpec((D2, D1), lambda j: (0, 0)),
            pl.BlockSpec((D2, D2), lambda j: (0, 0)),
            pl.BlockSpec((1, D2), lambda j: (0, 0)),
        ],
        out_specs=pl.BlockSpec((BN, D2), lambda j: (j, 0)),
        out_shape=jax.ShapeDtypeStruct((N, D2), jnp.float32),
    )(agg1, xs, disc, We, W1, b1r)


def _mm2_body(agg_ref, h1s_ref, disc_ref, W2_ref, b2_ref, Wc1_ref, bc1_ref,
              Wc2_ref, bc2_ref, h2_ref, log_ref):
    dcol = disc_ref[...]
    z = (agg_ref[...] + h1s_ref[...]) * dcol
    h2 = lax.dot_general(z, W2_ref[...], (((1,), (1,)), ((), ())),
                         preferred_element_type=jnp.float32) + b2_ref[...]
    h2_ref[...] = h2
    cc = lax.dot_general(h2, Wc1_ref[...], (((1,), (1,)), ((), ())),
                         preferred_element_type=jnp.float32)
    cc = jnp.maximum(cc + bc1_ref[...], 0.0)
    log_ref[...] = lax.dot_general(cc, Wc2_ref[...], (((1,), (1,)), ((), ())),
                                   preferred_element_type=jnp.float32) + bc2_ref[...]


def _mm2_call(agg2, h1s, disc, W2, b2r, Wc1, bc1r, Wc2p, bc2p):
    HC = D2 // 2
    return pl.pallas_call(
        _mm2_body,
        grid=(GRID,),
        in_specs=[
            pl.BlockSpec((BN, D2), lambda j: (j, 0)),
            pl.BlockSpec((BN, D2), lambda j: (j, 0)),
            pl.BlockSpec((BN, 1), lambda j: (j, 0)),
            pl.BlockSpec((D2, D2), lambda j: (0, 0)),
            pl.BlockSpec((1, D2), lambda j: (0, 0)),
            pl.BlockSpec((HC, D2), lambda j: (0, 0)),
            pl.BlockSpec((1, HC), lambda j: (0, 0)),
            pl.BlockSpec((16, HC), lambda j: (0, 0)),
            pl.BlockSpec((1, 16), lambda j: (0, 0)),
        ],
        out_specs=[
            pl.BlockSpec((BN, D2), lambda j: (j, 0)),
            pl.BlockSpec((BN, 16), lambda j: (j, 0)),
        ],
        out_shape=[
            jax.ShapeDtypeStruct((N, D2), jnp.float32),
            jax.ShapeDtypeStruct((N, 16), jnp.float32),
        ],
    )(agg2, h1s, disc, W2, b2r, Wc1, bc1r, Wc2p, bc2p)


@jax.jit
def kernel(x, edge_index, We, be, W1, b1, W2, b2, Wc1, bc1, Wc2, bc2):
    srcf = edge_index[0]
    dstf = edge_index[1]

    deg = _deg_call(dstf, jnp.zeros((NP,), jnp.float32))
    xs, disc = _scale_call(deg[:N].reshape(N, 1), x)

    FH1 = D1 // 16
    agg1f = _agg_call(srcf, dstf, xs.reshape(N * FH1, 16),
                      jnp.zeros((NACC, 16), jnp.float32),
                      _oidx_table(FH1), FH1, 1)
    agg1 = agg1f[:N * FH1].reshape(N, D1)

    h1s = _mm1_call(agg1, xs, disc, We, W1, b1.reshape(1, D2))

    FH2 = D2 // 16
    agg2f = _agg_call(srcf, dstf, h1s.reshape(N * FH2, 16),
                      jnp.zeros((NACC, 16), jnp.float32),
                      _oidx_table(FH2), FH2, 2)
    agg2 = agg2f[:N * FH2].reshape(N, D2)

    Wc2p = jnp.zeros((16, D2 // 2), jnp.float32).at[:10].set(Wc2)
    bc2p = jnp.zeros((1, 16), jnp.float32).at[:, :10].set(bc2)
    h2, logits16 = _mm2_call(agg2, h1s, disc, W2, b2.reshape(1, D2),
                             Wc1, bc1.reshape(1, D2 // 2), Wc2p, bc2p)
    return logits16[:, :10], h2
